# baseline (speedup 1.0000x reference)
import functools
import jax
import jax.numpy as jnp
from jax import lax
from jax.experimental import pallas as pl
from jax.experimental.pallas import tpu as pltpu
from jax.experimental.pallas import tpu_sc as plsc

_N_BOXES = 20000
_N_PAD = 20480
_NB = 160
_N_LABELS = 91
_KD = 100


def _cmp_desc(av, ai, bv, bi):
    return (av > bv) | ((av == bv) & (ai < bi))


def _cmpex(v, idx, s, dird):
    e = jax.lax.broadcasted_iota(jnp.int32, (v.shape[0], 1), 0)
    role_a = (e & s) == 0
    y_v = jnp.roll(v, -s, axis=0)
    z_v = jnp.roll(v, s, axis=0)
    y_i = jnp.roll(idx, -s, axis=0)
    z_i = jnp.roll(idx, s, axis=0)
    p_v = jnp.where(role_a, y_v, z_v)
    p_i = jnp.where(role_a, y_i, z_i)
    mine = _cmp_desc(v, idx, p_v, p_i)
    take_mine = mine == (dird == role_a)
    return jnp.where(take_mine, v, p_v), jnp.where(take_mine, idx, p_i)


def _sort128(v, idx, blk_odd):
    e = jax.lax.broadcasted_iota(jnp.int32, (128, 1), 0)
    for k in (2, 4, 8, 16, 32, 64, 128):
        dird = ((e & k) == 0) != blk_odd
        j = k // 2
        while j >= 1:
            v, idx = _cmpex(v, idx, j, dird)
            j //= 2
    return v, idx


def _sort2blocks(v, idx):
    e = jax.lax.broadcasted_iota(jnp.int32, (256, 1), 0)
    blk_odd = (e & 128) != 0
    el = e & 127
    for k in (2, 4, 8, 16, 32, 64, 128):
        dird = ((el & k) == 0) != blk_odd
        j = k // 2
        while j >= 1:
            v, idx = _cmpex(v, idx, j, dird)
            j //= 2
    return v, idx


def _clean128(v, idx, dird):
    j = 64
    while j >= 1:
        v, idx = _cmpex(v, idx, j, dird)
        j //= 2
    return v, idx


def _sel_body(s_ref, rc_ref, an_ref, ov_ref, oi_ref, od_ref, v_s, i_s):
    C = _N_LABELS

    def sort_pair(q, _):
        probs = jax.nn.sigmoid(s_ref[0, pl.ds(q * 256, 256), :])
        idx = jax.lax.broadcasted_iota(jnp.int32, (256, C), 0) + q * 256
        sv, si = _sort2blocks(probs, idx)
        v_s[2 * q] = sv[:128]
        i_s[2 * q] = si[:128]
        v_s[2 * q + 1] = sv[128:]
        i_s[2 * q + 1] = si[128:]
        return 0

    jax.lax.fori_loop(0, 78, sort_pair, 0)

    tail = jax.nn.sigmoid(s_ref[0, pl.ds(156 * 128, 32), :])
    tv = jnp.concatenate([tail, jnp.full((96, C), -1.0, jnp.float32)], axis=0)
    ti = jax.lax.broadcasted_iota(jnp.int32, (128, C), 0) + 156 * 128
    sv, si = _sort128(tv, ti, False)
    v_s[156] = sv
    i_s[156] = si

    ii = jax.lax.broadcasted_iota(jnp.int32, (128, C), 0)
    for b in (157, 158, 159):
        v_s[b] = jnp.full((128, C), -1.0, jnp.float32)
        i_s[b] = jnp.where(b % 2 == 1, 127 - ii, ii) + b * 128

    nb = _NB
    while nb > 1:
        p = nb // 2

        def merge_q(q, _):
            a_v, b_v = v_s[2 * q], v_s[2 * q + 1]
            a_i, b_i = i_s[2 * q], i_s[2 * q + 1]
            ka = _cmp_desc(a_v, a_i, b_v, b_i)
            m_v = jnp.where(ka, a_v, b_v)
            m_i = jnp.where(ka, a_i, b_i)
            m_v, m_i = _clean128(m_v, m_i, (q % 2) == 0)
            v_s[q] = m_v
            i_s[q] = m_i
            return 0

        jax.lax.fori_loop(0, p, merge_q, 0)
        if nb % 2 == 1:
            lv, li = v_s[2 * p], i_s[2 * p]
            if p % 2 == 1:
                lv, li = _clean128(lv, li, False)
            v_s[p] = lv
            i_s[p] = li
        nb = p + (nb % 2)

    ov_ref[0] = v_s[0]
    oi_ref[0] = i_s[0]

    ay1, ax1, ay2, ax2 = an_ref[0], an_ref[1], an_ref[2], an_ref[3]
    ty, tx, th, tw = rc_ref[0, 0], rc_ref[0, 1], rc_ref[0, 2], rc_ref[0, 3]
    ya = (ay1 + ay2) / 2.0
    xa = (ax1 + ax2) / 2.0
    ha = ay2 - ay1
    wa = ax2 - ax1
    ycenter = ty / 10.0 * ha + ya
    xcenter = tx / 10.0 * wa + xa
    h = jnp.exp(th / 5.0) * ha
    w = jnp.exp(tw / 5.0) * wa
    od_ref[0, 0] = jnp.clip(ycenter - h / 2.0, 0.0, 512.0)
    od_ref[0, 1] = jnp.clip(xcenter - w / 2.0, 0.0, 512.0)
    od_ref[0, 2] = jnp.clip(ycenter + h / 2.0, 0.0, 512.0)
    od_ref[0, 3] = jnp.clip(xcenter + w / 2.0, 0.0, 512.0)


def _select_topk(scores, rc_pl, an_pl):
    B = scores.shape[0]
    return pl.pallas_call(
        _sel_body,
        out_shape=(
            jax.ShapeDtypeStruct((B, 128, _N_LABELS), jnp.float32),
            jax.ShapeDtypeStruct((B, 128, _N_LABELS), jnp.int32),
            jax.ShapeDtypeStruct((B, 4, _NB, 128), jnp.float32),
        ),
        grid=(B,),
        in_specs=[
            pl.BlockSpec((1, _N_BOXES, _N_LABELS), lambda b: (b, 0, 0)),
            pl.BlockSpec((1, 4, _NB, 128), lambda b: (b, 0, 0, 0)),
            pl.BlockSpec((4, _NB, 128), lambda b: (0, 0, 0)),
        ],
        out_specs=(
            pl.BlockSpec((1, 128, _N_LABELS), lambda b: (b, 0, 0)),
            pl.BlockSpec((1, 128, _N_LABELS), lambda b: (b, 0, 0)),
            pl.BlockSpec((1, 4, _NB, 128), lambda b: (b, 0, 0, 0)),
        ),
        scratch_shapes=[
            pltpu.VMEM((_NB, 128, _N_LABELS), jnp.float32),
            pltpu.VMEM((_NB, 128, _N_LABELS), jnp.int32),
        ],
    )(scores, rc_pl, an_pl)


_NG = 73728
_GW = _NG // 32
_GC = 128


_NPI = 9216
_WPI = 4
_GW = _NPI // _WPI


def _sc_gather(planes, idx_local):
    B = planes.shape[0]
    mesh = plsc.VectorSubcoreMesh(core_axis_name="c", subcore_axis_name="s")

    @functools.partial(
        pl.kernel,
        mesh=mesh,
        out_type=jax.ShapeDtypeStruct((B, 4, _NPI), jnp.float32),
        compiler_params=pltpu.CompilerParams(needs_layout_passes=False),
        scratch_types=[
            pltpu.VMEM((_N_PAD,), jnp.float32),
            pltpu.VMEM((_N_PAD,), jnp.float32),
            pltpu.VMEM((_N_PAD,), jnp.float32),
            pltpu.VMEM((_N_PAD,), jnp.float32),
            pltpu.VMEM((_GW,), jnp.int32),
            pltpu.VMEM((4, _GW), jnp.float32),
        ],
    )
    def gk(planes_hbm, idx_hbm, out_hbm, p0, p1, p2, p3, idx_v, out_v):
        wid = lax.axis_index("s") * 2 + lax.axis_index("c")
        img = wid // _WPI
        part = wid % _WPI
        pls = (p0, p1, p2, p3)
        for k in range(4):
            pltpu.sync_copy(planes_hbm.at[img].at[k], pls[k])
        pltpu.sync_copy(idx_hbm.at[img].at[pl.ds(part * _GW, _GW)], idx_v)

        def step(t, _):
            iv = idx_v[pl.ds(t * 16, 16)]
            for k in range(4):
                row = plsc.load_gather(pls[k], [iv])
                out_v[k, pl.ds(t * 16, 16)] = row
            return 0

        lax.fori_loop(0, _GW // 16, step, 0)
        pltpu.sync_copy(out_v, out_hbm.at[img].at[:, pl.ds(part * _GW, _GW)])

    return gk(planes, idx_local)


def _nms_body(v_ref, bx_ref, ob_ref, os_ref, ol_ref, on_ref, iou_s, val_s):
    vals = v_ref[0, :_KD]
    valid = vals > 0.05

    y1, x1, y2, x2 = bx_ref[0, 0], bx_ref[0, 1], bx_ref[0, 2], bx_ref[0, 3]
    area = (y2 - y1) * (x2 - x1)
    iy = jnp.maximum(0.0, jnp.minimum(y2[:, None, :], y2[None, :, :])
                     - jnp.maximum(y1[:, None, :], y1[None, :, :]))
    ix = jnp.maximum(0.0, jnp.minimum(x2[:, None, :], x2[None, :, :])
                     - jnp.maximum(x1[:, None, :], x1[None, :, :]))
    inter = iy * ix
    union = area[:, None, :] + area[None, :, :] - inter
    iou_s[...] = inter / jnp.maximum(union, 1e-8)
    val_s[...] = jnp.where(valid, 1.0, 0.0).reshape(_KD, 1, _N_LABELS)

    row_iota = jax.lax.broadcasted_iota(jnp.int32, (_KD, _N_LABELS), 0)

    def body(i, keep):
        row = iou_s[i]
        sup_m = jnp.where((row > 0.5) & (row_iota < i), keep, 0.0)
        sup = jnp.max(sup_m, axis=0, keepdims=True) > 0.0
        vrow = val_s[i] > 0.0
        newrow = jnp.where(vrow & jnp.logical_not(sup), 1.0, 0.0)
        return jnp.where(row_iota == i, newrow, keep)

    keep = jax.lax.fori_loop(0, _KD, body, jnp.zeros((_KD, _N_LABELS), jnp.float32))

    cand = jnp.where(keep > 0.0, vals, -1e9)
    flat = row_iota + jax.lax.broadcasted_iota(jnp.int32, (_KD, _N_LABELS), 1) * _KD

    def ext_body(t, carry):
        cand, oby, obx, oby2, obx2, osc, ola, num = carry
        m = jnp.max(cand)
        mi = jnp.min(jnp.where(cand == m, flat, jnp.int32(2 ** 30)))
        sel = ((flat == mi) & (cand == m)).astype(jnp.float32)
        b_y1 = jnp.sum(sel * y1)
        b_x1 = jnp.sum(sel * x1)
        b_y2 = jnp.sum(sel * y2)
        b_x2 = jnp.sum(sel * x2)
        lab = (mi // _KD).astype(jnp.float32)
        vld = m > 0.05
        t_iota = jax.lax.iota(jnp.int32, _KD)
        hit = t_iota == t
        oby = jnp.where(hit, jnp.where(vld, b_y1, 0.0), oby)
        obx = jnp.where(hit, jnp.where(vld, b_x1, 0.0), obx)
        oby2 = jnp.where(hit, jnp.where(vld, b_y2, 0.0), oby2)
        obx2 = jnp.where(hit, jnp.where(vld, b_x2, 0.0), obx2)
        osc = jnp.where(hit, jnp.where(vld, m, 0.0), osc)
        ola = jnp.where(hit, jnp.where(vld, lab, 0.0), ola)
        num = num + jnp.where(vld, 1, 0)
        cand = jnp.where(sel > 0.0, -2e9, cand)
        return cand, oby, obx, oby2, obx2, osc, ola, num

    z = jnp.zeros((_KD,), jnp.float32)
    carry = (cand, z, z, z, z, z, z, jnp.int32(0))
    _, oby, obx, oby2, obx2, osc, ola, num = jax.lax.fori_loop(0, _KD, ext_body, carry)

    ob_ref[0] = jnp.stack([oby, obx, oby2, obx2], axis=-1)
    os_ref[0, 0] = osc
    ol_ref[0, 0] = ola
    on_ref[0] = jnp.broadcast_to(num, (1, 1))


def _nms_merge(vals, boxes_pl):
    B = vals.shape[0]
    return pl.pallas_call(
        _nms_body,
        out_shape=(
            jax.ShapeDtypeStruct((B, _KD, 4), jnp.float32),
            jax.ShapeDtypeStruct((B, 1, _KD), jnp.float32),
            jax.ShapeDtypeStruct((B, 1, _KD), jnp.float32),
            jax.ShapeDtypeStruct((B, 1, 1), jnp.int32),
        ),
        grid=(B,),
        in_specs=[
            pl.BlockSpec((1, 128, _N_LABELS), lambda b: (b, 0, 0)),
            pl.BlockSpec((1, 4, _KD, _N_LABELS), lambda b: (b, 0, 0, 0)),
        ],
        out_specs=(
            pl.BlockSpec((1, _KD, 4), lambda b: (b, 0, 0)),
            pl.BlockSpec((1, 1, _KD), lambda b: (b, 0, 0)),
            pl.BlockSpec((1, 1, _KD), lambda b: (b, 0, 0)),
            pl.BlockSpec((1, 1, 1), lambda b: (b, 0, 0)),
        ),
        scratch_shapes=[
            pltpu.VMEM((_KD, _KD, _N_LABELS), jnp.float32),
            pltpu.VMEM((_KD, 1, _N_LABELS), jnp.float32),
        ],
    )(vals, boxes_pl)


def kernel(rel_codes, scores, anchors):
    B = scores.shape[0]
    pad = _N_PAD - _N_BOXES
    rc_pl = jnp.pad(rel_codes.transpose(0, 2, 1), ((0, 0), (0, 0), (0, pad)))
    rc_pl = rc_pl.reshape(B, 4, _NB, 128)
    an_pl = jnp.pad(anchors.transpose(1, 0), ((0, 0), (0, pad)), constant_values=1.0)
    an_pl = an_pl.reshape(4, _NB, 128)

    ov, oi, od = _select_topk(scores, rc_pl, an_pl)

    planes = od.reshape(B, 4, _N_PAD)
    flat = oi[:, :_KD, :].reshape(B, _KD * _N_LABELS)
    flat = jnp.pad(flat, ((0, 0), (0, _NPI - _KD * _N_LABELS)))
    gathered = _sc_gather(planes, flat)
    boxes_pl = gathered[:, :, :_KD * _N_LABELS].reshape(B, 4, _KD, _N_LABELS)

    fb, fs, fl, fn = _nms_merge(ov, boxes_pl)
    return fb, fs.reshape(B, _KD), fl.reshape(B, _KD), fn.reshape(B)


# speedup vs baseline: 1.0234x; 1.0234x over previous
import functools
import jax
import jax.numpy as jnp
from jax import lax
from jax.experimental import pallas as pl
from jax.experimental.pallas import tpu as pltpu
from jax.experimental.pallas import tpu_sc as plsc

_N_BOXES = 20000
_N_PAD = 20480
_NB = 160
_N_LABELS = 91
_KD = 100


def _cmp_desc(av, ai, bv, bi):
    return (av > bv) | ((av == bv) & (ai < bi))


def _cmpex(v, idx, s, dird):
    e = jax.lax.broadcasted_iota(jnp.int32, (128, 1), 0)
    role_a = (e & s) == 0
    y_v = jnp.roll(v, -s, axis=0)
    z_v = jnp.roll(v, s, axis=0)
    y_i = jnp.roll(idx, -s, axis=0)
    z_i = jnp.roll(idx, s, axis=0)
    p_v = jnp.where(role_a, y_v, z_v)
    p_i = jnp.where(role_a, y_i, z_i)
    mine = _cmp_desc(v, idx, p_v, p_i)
    take_mine = mine == (dird == role_a)
    return jnp.where(take_mine, v, p_v), jnp.where(take_mine, idx, p_i)


def _sort128(v, idx, blk_odd):
    e = jax.lax.broadcasted_iota(jnp.int32, (128, 1), 0)
    for k in (2, 4, 8, 16, 32, 64, 128):
        dird = ((e & k) == 0) != blk_odd
        j = k // 2
        while j >= 1:
            v, idx = _cmpex(v, idx, j, dird)
            j //= 2
    return v, idx


def _clean128(v, idx, dird):
    j = 64
    while j >= 1:
        v, idx = _cmpex(v, idx, j, dird)
        j //= 2
    return v, idx


def _sel_body(s_ref, rc_ref, an_ref, ov_ref, oi_ref, od_ref, v_s, i_s):
    C = _N_LABELS

    def sort_full_block(b, _):
        probs = jax.nn.sigmoid(s_ref[0, pl.ds(b * 128, 128), :])
        idx = jax.lax.broadcasted_iota(jnp.int32, (128, C), 0) + b * 128
        sv, si = _sort128(probs, idx, (b % 2) == 1)
        v_s[b] = sv
        i_s[b] = si
        return 0

    jax.lax.fori_loop(0, 156, sort_full_block, 0)

    tail = jax.nn.sigmoid(s_ref[0, pl.ds(156 * 128, 32), :])
    tv = jnp.concatenate([tail, jnp.full((96, C), -1.0, jnp.float32)], axis=0)
    ti = jax.lax.broadcasted_iota(jnp.int32, (128, C), 0) + 156 * 128
    sv, si = _sort128(tv, ti, False)
    v_s[156] = sv
    i_s[156] = si

    ii = jax.lax.broadcasted_iota(jnp.int32, (128, C), 0)
    for b in (157, 158, 159):
        v_s[b] = jnp.full((128, C), -1.0, jnp.float32)
        i_s[b] = jnp.where(b % 2 == 1, 127 - ii, ii) + b * 128

    nb = _NB
    while nb > 1:
        p = nb // 2

        def merge_q(q, _):
            a_v, b_v = v_s[2 * q], v_s[2 * q + 1]
            a_i, b_i = i_s[2 * q], i_s[2 * q + 1]
            ka = _cmp_desc(a_v, a_i, b_v, b_i)
            m_v = jnp.where(ka, a_v, b_v)
            m_i = jnp.where(ka, a_i, b_i)
            m_v, m_i = _clean128(m_v, m_i, (q % 2) == 0)
            v_s[q] = m_v
            i_s[q] = m_i
            return 0

        jax.lax.fori_loop(0, p, merge_q, 0)
        if nb % 2 == 1:
            lv, li = v_s[2 * p], i_s[2 * p]
            if p % 2 == 1:
                lv, li = _clean128(lv, li, False)
            v_s[p] = lv
            i_s[p] = li
        nb = p + (nb % 2)

    ov_ref[0] = v_s[0]
    oi_ref[0] = i_s[0]

    ay1, ax1, ay2, ax2 = an_ref[0], an_ref[1], an_ref[2], an_ref[3]
    ty, tx, th, tw = rc_ref[0, 0], rc_ref[0, 1], rc_ref[0, 2], rc_ref[0, 3]
    ya = (ay1 + ay2) / 2.0
    xa = (ax1 + ax2) / 2.0
    ha = ay2 - ay1
    wa = ax2 - ax1
    ycenter = ty / 10.0 * ha + ya
    xcenter = tx / 10.0 * wa + xa
    h = jnp.exp(th / 5.0) * ha
    w = jnp.exp(tw / 5.0) * wa
    od_ref[0, 0] = jnp.clip(ycenter - h / 2.0, 0.0, 512.0)
    od_ref[0, 1] = jnp.clip(xcenter - w / 2.0, 0.0, 512.0)
    od_ref[0, 2] = jnp.clip(ycenter + h / 2.0, 0.0, 512.0)
    od_ref[0, 3] = jnp.clip(xcenter + w / 2.0, 0.0, 512.0)


def _select_topk(scores, rc_pl, an_pl):
    B = scores.shape[0]
    return pl.pallas_call(
        _sel_body,
        out_shape=(
            jax.ShapeDtypeStruct((B, 128, _N_LABELS), jnp.float32),
            jax.ShapeDtypeStruct((B, 128, _N_LABELS), jnp.int32),
            jax.ShapeDtypeStruct((B, 4, _NB, 128), jnp.float32),
        ),
        grid=(B,),
        in_specs=[
            pl.BlockSpec((1, _N_BOXES, _N_LABELS), lambda b: (b, 0, 0)),
            pl.BlockSpec((1, 4, _NB, 128), lambda b: (b, 0, 0, 0)),
            pl.BlockSpec((4, _NB, 128), lambda b: (0, 0, 0)),
        ],
        out_specs=(
            pl.BlockSpec((1, 128, _N_LABELS), lambda b: (b, 0, 0)),
            pl.BlockSpec((1, 128, _N_LABELS), lambda b: (b, 0, 0)),
            pl.BlockSpec((1, 4, _NB, 128), lambda b: (b, 0, 0, 0)),
        ),
        scratch_shapes=[
            pltpu.VMEM((_NB, 128, _N_LABELS), jnp.float32),
            pltpu.VMEM((_NB, 128, _N_LABELS), jnp.int32),
        ],
    )(scores, rc_pl, an_pl)


_NPI = 9216
_WPI = 4
_GW = _NPI // _WPI


def _sc_gather(planes, idx_local):
    B = planes.shape[0]
    mesh = plsc.VectorSubcoreMesh(core_axis_name="c", subcore_axis_name="s")

    @functools.partial(
        pl.kernel,
        mesh=mesh,
        out_type=jax.ShapeDtypeStruct((B, 4, _NPI), jnp.float32),
        compiler_params=pltpu.CompilerParams(needs_layout_passes=False),
        scratch_types=[
            pltpu.VMEM((_N_PAD,), jnp.float32),
            pltpu.VMEM((_N_PAD,), jnp.float32),
            pltpu.VMEM((_N_PAD,), jnp.float32),
            pltpu.VMEM((_N_PAD,), jnp.float32),
            pltpu.VMEM((_GW,), jnp.int32),
            pltpu.VMEM((4, _GW), jnp.float32),
        ],
    )
    def gk(planes_hbm, idx_hbm, out_hbm, p0, p1, p2, p3, idx_v, out_v):
        wid = lax.axis_index("s") * 2 + lax.axis_index("c")
        img = wid // _WPI
        part = wid % _WPI
        pls = (p0, p1, p2, p3)
        for k in range(4):
            pltpu.sync_copy(planes_hbm.at[img].at[k], pls[k])
        pltpu.sync_copy(idx_hbm.at[img].at[pl.ds(part * _GW, _GW)], idx_v)

        def step(t, _):
            iv = idx_v[pl.ds(t * 16, 16)]
            for k in range(4):
                row = plsc.load_gather(pls[k], [iv])
                out_v[k, pl.ds(t * 16, 16)] = row
            return 0

        lax.fori_loop(0, _GW // 16, step, 0)
        pltpu.sync_copy(out_v, out_hbm.at[img].at[:, pl.ds(part * _GW, _GW)])

    return gk(planes, idx_local)


def _nms_body(v_ref, bx_ref, ob_ref, os_ref, ol_ref, on_ref, iou_s, val_s):
    vals = v_ref[0, :_KD]
    valid = vals > 0.05

    y1, x1, y2, x2 = bx_ref[0, 0], bx_ref[0, 1], bx_ref[0, 2], bx_ref[0, 3]
    area = (y2 - y1) * (x2 - x1)
    iy = jnp.maximum(0.0, jnp.minimum(y2[:, None, :], y2[None, :, :])
                     - jnp.maximum(y1[:, None, :], y1[None, :, :]))
    ix = jnp.maximum(0.0, jnp.minimum(x2[:, None, :], x2[None, :, :])
                     - jnp.maximum(x1[:, None, :], x1[None, :, :]))
    inter = iy * ix
    union = area[:, None, :] + area[None, :, :] - inter
    iou_s[...] = inter / jnp.maximum(union, 1e-8)
    val_s[...] = jnp.where(valid, 1.0, 0.0).reshape(_KD, 1, _N_LABELS)

    row_iota = jax.lax.broadcasted_iota(jnp.int32, (_KD, _N_LABELS), 0)

    def body(i, keep):
        row = iou_s[i]
        sup_m = jnp.where((row > 0.5) & (row_iota < i), keep, 0.0)
        sup = jnp.max(sup_m, axis=0, keepdims=True) > 0.0
        vrow = val_s[i] > 0.0
        newrow = jnp.where(vrow & jnp.logical_not(sup), 1.0, 0.0)
        return jnp.where(row_iota == i, newrow, keep)

    keep = jax.lax.fori_loop(0, _KD, body, jnp.zeros((_KD, _N_LABELS), jnp.float32))

    cand = jnp.where(keep > 0.0, vals, -1e9)
    flat = row_iota + jax.lax.broadcasted_iota(jnp.int32, (_KD, _N_LABELS), 1) * _KD

    def ext_body(t, carry):
        cand, oby, obx, oby2, obx2, osc, ola, num = carry
        m = jnp.max(cand)
        mi = jnp.min(jnp.where(cand == m, flat, jnp.int32(2 ** 30)))
        sel = ((flat == mi) & (cand == m)).astype(jnp.float32)
        b_y1 = jnp.sum(sel * y1)
        b_x1 = jnp.sum(sel * x1)
        b_y2 = jnp.sum(sel * y2)
        b_x2 = jnp.sum(sel * x2)
        lab = (mi // _KD).astype(jnp.float32)
        vld = m > 0.05
        t_iota = jax.lax.iota(jnp.int32, _KD)
        hit = t_iota == t
        oby = jnp.where(hit, jnp.where(vld, b_y1, 0.0), oby)
        obx = jnp.where(hit, jnp.where(vld, b_x1, 0.0), obx)
        oby2 = jnp.where(hit, jnp.where(vld, b_y2, 0.0), oby2)
        obx2 = jnp.where(hit, jnp.where(vld, b_x2, 0.0), obx2)
        osc = jnp.where(hit, jnp.where(vld, m, 0.0), osc)
        ola = jnp.where(hit, jnp.where(vld, lab, 0.0), ola)
        num = num + jnp.where(vld, 1, 0)
        cand = jnp.where(sel > 0.0, -2e9, cand)
        return cand, oby, obx, oby2, obx2, osc, ola, num

    z = jnp.zeros((_KD,), jnp.float32)
    carry = (cand, z, z, z, z, z, z, jnp.int32(0))
    _, oby, obx, oby2, obx2, osc, ola, num = jax.lax.fori_loop(0, _KD, ext_body, carry)

    ob_ref[0] = jnp.stack([oby, obx, oby2, obx2], axis=-1)
    os_ref[0, 0] = osc
    ol_ref[0, 0] = ola
    on_ref[0] = jnp.broadcast_to(num, (1, 1))


def _nms_merge(vals, boxes_pl):
    B = vals.shape[0]
    return pl.pallas_call(
        _nms_body,
        out_shape=(
            jax.ShapeDtypeStruct((B, _KD, 4), jnp.float32),
            jax.ShapeDtypeStruct((B, 1, _KD), jnp.float32),
            jax.ShapeDtypeStruct((B, 1, _KD), jnp.float32),
            jax.ShapeDtypeStruct((B, 1, 1), jnp.int32),
        ),
        grid=(B,),
        in_specs=[
            pl.BlockSpec((1, 128, _N_LABELS), lambda b: (b, 0, 0)),
            pl.BlockSpec((1, 4, _KD, _N_LABELS), lambda b: (b, 0, 0, 0)),
        ],
        out_specs=(
            pl.BlockSpec((1, _KD, 4), lambda b: (b, 0, 0)),
            pl.BlockSpec((1, 1, _KD), lambda b: (b, 0, 0)),
            pl.BlockSpec((1, 1, _KD), lambda b: (b, 0, 0)),
            pl.BlockSpec((1, 1, 1), lambda b: (b, 0, 0)),
        ),
        scratch_shapes=[
            pltpu.VMEM((_KD, _KD, _N_LABELS), jnp.float32),
            pltpu.VMEM((_KD, 1, _N_LABELS), jnp.float32),
        ],
    )(vals, boxes_pl)


def kernel(rel_codes, scores, anchors):
    B = scores.shape[0]
    pad = _N_PAD - _N_BOXES
    rc_pl = jnp.pad(rel_codes.transpose(0, 2, 1), ((0, 0), (0, 0), (0, pad)))
    rc_pl = rc_pl.reshape(B, 4, _NB, 128)
    an_pl = jnp.pad(anchors.transpose(1, 0), ((0, 0), (0, pad)), constant_values=1.0)
    an_pl = an_pl.reshape(4, _NB, 128)

    ov, oi, od = _select_topk(scores, rc_pl, an_pl)

    planes = od.reshape(B, 4, _N_PAD)
    flat = oi[:, :_KD, :].reshape(B, _KD * _N_LABELS)
    flat = jnp.pad(flat, ((0, 0), (0, _NPI - _KD * _N_LABELS)))
    gathered = _sc_gather(planes, flat)
    boxes_pl = gathered[:, :, :_KD * _N_LABELS].reshape(B, 4, _KD, _N_LABELS)

    fb, fs, fl, fn = _nms_merge(ov, boxes_pl)
    return fb, fs.reshape(B, _KD), fl.reshape(B, _KD), fn.reshape(B)
